# baseline (speedup 1.0000x reference)
import functools

import jax
import jax.numpy as jnp
from jax import lax
from jax.experimental import pallas as pl
from jax.experimental.pallas import tpu as pltpu
from jax.experimental.pallas import tpu_sc as plsc

N_NODES = 10000
N_EDGES = 320000
D_FEAT = 128
LANES = 16
NUM_CORES = 2
NUM_SUBCORES = 16
DH = D_FEAT // NUM_CORES
EDGES_PER_TILE = N_EDGES // NUM_SUBCORES
CHUNK = 128
NCH = -(-EDGES_PER_TILE // CHUNK)
EPT_PAD = NCH * CHUNK
PAD = EPT_PAD - EDGES_PER_TILE
ROWS_PER_TILE = 624
ZROWS = 48
SLOPE = 0.5


def _sc_body(row_hbm, col_hbm, val_hbm, emb_hbm, out_hbm,
             col0, col1, row0, row1, val0, val1, rows0, rows1, zbuf, semb, acc,
             gsem0, gsem1, isem0, isem1, csem0, csem1):
    c = lax.axis_index("c")
    s = lax.axis_index("s")
    ebase = s * EPT_PAD
    rows_b = (rows0, rows1)
    col_b = (col0, col1)
    row_b = (row0, row1)
    val_b = (val0, val1)
    gsem_b = (gsem0, gsem1)
    isem_b = (isem0, isem1)
    csem_b = (csem0, csem1)
    rbase = s * ROWS_PER_TILE
    body = NUM_SUBCORES * ROWS_PER_TILE
    rem = N_NODES - body
    last = NUM_SUBCORES - 1

    pltpu.sync_copy(emb_hbm.at[c, pl.ds(rbase, ROWS_PER_TILE)],
                    semb.at[pl.ds(rbase, ROWS_PER_TILE)])
    @pl.when(s == last)
    def _stage_rem():
        pltpu.sync_copy(emb_hbm.at[c, pl.ds(body, rem)],
                        semb.at[pl.ds(body, rem)])

    def _zero_z(i, _):
        for j in range(DH // LANES):
            zbuf[i, pl.ds(j * LANES, LANES)] = jnp.zeros((LANES,), jnp.float32)
        return 0
    lax.fori_loop(0, ZROWS, _zero_z, 0)
    for t in range(ROWS_PER_TILE // ZROWS):
        pltpu.sync_copy(zbuf, acc.at[pl.ds(rbase + t * ZROWS, ZROWS)])
    @pl.when(s == last)
    def _zero_rem():
        pltpu.sync_copy(zbuf.at[pl.ds(0, rem)], acc.at[pl.ds(body, rem)])
    plsc.subcore_barrier()

    def _issue_col(k, b):
        pltpu.async_copy(col_hbm.at[pl.ds(ebase + k * CHUNK, CHUNK)],
                         col_b[b], csem_b[b])

    def _wait_col(k, b):
        pltpu.make_async_copy(col_hbm.at[pl.ds(ebase + k * CHUNK, CHUNK)],
                              col_b[b], csem_b[b]).wait()

    def _issue(k, b):
        pltpu.async_copy(semb.at[col_b[b]], rows_b[b], gsem_b[b])
        pltpu.async_copy(row_hbm.at[pl.ds(ebase + k * CHUNK, CHUNK)],
                         row_b[b], isem_b[b])
        pltpu.async_copy(val_hbm.at[pl.ds(ebase + k * CHUNK, CHUNK)],
                         val_b[b], isem_b[b])

    def _wait(k, b):
        pltpu.make_async_copy(semb.at[col_b[b]], rows_b[b], gsem_b[b]).wait()
        pltpu.make_async_copy(row_hbm.at[pl.ds(ebase + k * CHUNK, CHUNK)],
                              row_b[b], isem_b[b]).wait()
        pltpu.make_async_copy(val_hbm.at[pl.ds(ebase + k * CHUNK, CHUNK)],
                              val_b[b], isem_b[b]).wait()

    def _scale(b):
        rowsb = rows_b[b]
        valb = val_b[b]

        def _group(g, _):
            val16 = valb[pl.ds(g * LANES, LANES)]
            for e_loc in range(LANES):
                bvec = jnp.take_along_axis(
                    val16, jnp.full((LANES,), e_loc, jnp.int32), axis=0)
                e = g * LANES + e_loc
                for j in range(DH // LANES):
                    sl = pl.ds(j * LANES, LANES)
                    rowsb[e, sl] = rowsb[e, sl] * bvec
            return 0
        lax.fori_loop(0, CHUNK // LANES, _group, 0)

    _issue_col(0, 0)
    _issue_col(1, 1)
    _wait_col(0, 0)
    _issue(0, 0)

    def _pair(i, _):
        for b in range(2):
            k = i * 2 + b

            @pl.when(k < NCH)
            def _process():
                _wait(k, b)

                @pl.when(k + 2 < NCH)
                def _prefetch_col():
                    _issue_col(k + 2, b)

                @pl.when(k + 1 < NCH)
                def _prefetch():
                    _wait_col(k + 1, 1 - b)
                    _issue(k + 1, 1 - b)
                _scale(b)
                pltpu.sync_copy(rows_b[b], acc.at[row_b[b]], add=True)
        return 0
    lax.fori_loop(0, (NCH + 1) // 2, _pair, 0)
    plsc.subcore_barrier()

    pltpu.sync_copy(acc.at[pl.ds(rbase, ROWS_PER_TILE)],
                    out_hbm.at[c, pl.ds(rbase, ROWS_PER_TILE)])
    @pl.when(s == last)
    def _write_rem():
        pltpu.sync_copy(acc.at[pl.ds(body, rem)], out_hbm.at[c, pl.ds(body, rem)])


@functools.partial(
    pl.kernel,
    out_type=jax.ShapeDtypeStruct((NUM_CORES, N_NODES, DH), jnp.float32),
    mesh=plsc.VectorSubcoreMesh(core_axis_name="c", subcore_axis_name="s"),
    compiler_params=pltpu.CompilerParams(use_tc_tiling_on_sc=False),
    scratch_types=[
        pltpu.VMEM((CHUNK,), jnp.int32),
        pltpu.VMEM((CHUNK,), jnp.int32),
        pltpu.VMEM((CHUNK,), jnp.int32),
        pltpu.VMEM((CHUNK,), jnp.int32),
        pltpu.VMEM((CHUNK,), jnp.float32),
        pltpu.VMEM((CHUNK,), jnp.float32),
        pltpu.VMEM((CHUNK, DH), jnp.float32),
        pltpu.VMEM((CHUNK, DH), jnp.float32),
        pltpu.VMEM((ZROWS, DH), jnp.float32),
        pltpu.VMEM_SHARED((N_NODES, DH), jnp.float32),
        pltpu.VMEM_SHARED((N_NODES, DH), jnp.float32),
        pltpu.SemaphoreType.DMA,
        pltpu.SemaphoreType.DMA,
        pltpu.SemaphoreType.DMA,
        pltpu.SemaphoreType.DMA,
        pltpu.SemaphoreType.DMA,
        pltpu.SemaphoreType.DMA,
    ],
)
def _sc_spmm(row_hbm, col_hbm, val_hbm, emb_hbm, out_hbm, *scratch):
    _sc_body(row_hbm, col_hbm, val_hbm, emb_hbm, out_hbm, *scratch)


def _combine_body(p_ref, o_ref):
    for h in range(NUM_CORES):
        x = p_ref[h]
        o_ref[:, h * DH:(h + 1) * DH] = jnp.where(x >= 0, x, SLOPE * x)


def _combine(halves):
    blk = 1000
    return pl.pallas_call(
        _combine_body,
        grid=(N_NODES // blk,),
        in_specs=[pl.BlockSpec((NUM_CORES, blk, DH), lambda i: (0, i, 0))],
        out_specs=pl.BlockSpec((blk, D_FEAT), lambda i: (i, 0)),
        out_shape=jax.ShapeDtypeStruct((N_NODES, D_FEAT), jnp.float32),
    )(halves)


def kernel(adj_indices, adj_values, embeds):
    idx = adj_indices.astype(jnp.int32)
    pad2 = ((0, 0), (0, PAD))
    row1 = jnp.pad(idx[0].reshape(NUM_SUBCORES, EDGES_PER_TILE), pad2).reshape(-1)
    col1 = jnp.pad(idx[1].reshape(NUM_SUBCORES, EDGES_PER_TILE), pad2).reshape(-1)
    val1 = jnp.pad(adj_values.reshape(NUM_SUBCORES, EDGES_PER_TILE), pad2).reshape(-1)
    embh = embeds.reshape(N_NODES, NUM_CORES, DH).transpose(1, 0, 2)
    halves = _sc_spmm(row1, col1, val1, embh)
    return _combine(halves)


# speedup vs baseline: 1.2483x; 1.2483x over previous
import functools

import jax
import jax.numpy as jnp
from jax import lax
from jax.experimental import pallas as pl
from jax.experimental.pallas import tpu as pltpu
from jax.experimental.pallas import tpu_sc as plsc

N_NODES = 10000
N_EDGES = 320000
D_FEAT = 128
LANES = 16
NUM_CORES = 2
NUM_SUBCORES = 16
NUM_TILES = NUM_CORES * NUM_SUBCORES
DH = D_FEAT // 2
EDGES_PER_TILE = N_EDGES // NUM_TILES
CHUNK = 32
NCH = 314
EPT_PAD = NCH * CHUNK
PAD = EPT_PAD - EDGES_PER_TILE
ROWS_PER_TILE = 624
SLOPE = 0.5
HI_MASK = -65536


def _sc_body(row_hbm, col_hbm, val_hbm, emb_hbm, out_hbm,
             col0, col1, row0, row1, val0, val1, rows0, rows1, scl, semb, acc,
             gsem0, gsem1, isem0, isem1, csem0, csem1):
    c = lax.axis_index("c")
    s = lax.axis_index("s")
    wid = c * NUM_SUBCORES + s
    ebase = wid * EPT_PAD
    rows_b = (rows0, rows1)
    col_b = (col0, col1)
    row_b = (row0, row1)
    val_b = (val0, val1)
    gsem_b = (gsem0, gsem1)
    isem_b = (isem0, isem1)
    csem_b = (csem0, csem1)
    rbase = s * ROWS_PER_TILE
    body = NUM_SUBCORES * ROWS_PER_TILE
    rem = N_NODES - body
    last = NUM_SUBCORES - 1

    pltpu.sync_copy(emb_hbm.at[pl.ds(rbase, ROWS_PER_TILE)],
                    semb.at[pl.ds(rbase, ROWS_PER_TILE)])
    @pl.when(s == last)
    def _stage_rem():
        pltpu.sync_copy(emb_hbm.at[pl.ds(body, rem)], semb.at[pl.ds(body, rem)])

    def _zero_z(i, _):
        for j in range(D_FEAT // LANES):
            scl[i, pl.ds(j * LANES, LANES)] = jnp.zeros((LANES,), jnp.float32)
        return 0
    lax.fori_loop(0, CHUNK, _zero_z, 0)
    for t in range(ROWS_PER_TILE // CHUNK):
        pltpu.sync_copy(scl, acc.at[pl.ds(rbase + t * CHUNK, CHUNK)])
    t_rem = ROWS_PER_TILE - (ROWS_PER_TILE // CHUNK) * CHUNK
    pltpu.sync_copy(scl.at[pl.ds(0, t_rem)],
                    acc.at[pl.ds(rbase + ROWS_PER_TILE - t_rem, t_rem)])
    @pl.when(s == last)
    def _zero_rem():
        pltpu.sync_copy(scl.at[pl.ds(0, rem)], acc.at[pl.ds(body, rem)])
    plsc.subcore_barrier()

    def _issue_col(k, b):
        pltpu.async_copy(col_hbm.at[pl.ds(ebase + k * CHUNK, CHUNK)],
                         col_b[b], csem_b[b])

    def _wait_col(k, b):
        pltpu.make_async_copy(col_hbm.at[pl.ds(ebase + k * CHUNK, CHUNK)],
                              col_b[b], csem_b[b]).wait()

    def _issue(k, b):
        pltpu.async_copy(semb.at[col_b[b]], rows_b[b], gsem_b[b])
        pltpu.async_copy(row_hbm.at[pl.ds(ebase + k * CHUNK, CHUNK)],
                         row_b[b], isem_b[b])
        pltpu.async_copy(val_hbm.at[pl.ds(ebase + k * CHUNK, CHUNK)],
                         val_b[b], isem_b[b])

    def _wait(k, b):
        pltpu.make_async_copy(semb.at[col_b[b]], rows_b[b], gsem_b[b]).wait()
        pltpu.make_async_copy(row_hbm.at[pl.ds(ebase + k * CHUNK, CHUNK)],
                              row_b[b], isem_b[b]).wait()
        pltpu.make_async_copy(val_hbm.at[pl.ds(ebase + k * CHUNK, CHUNK)],
                              val_b[b], isem_b[b]).wait()

    def _unpack_scale(b):
        rowsb = rows_b[b]
        valb = val_b[b]

        def _group(g, _):
            val16 = valb[pl.ds(g * LANES, LANES)]
            for e_loc in range(LANES):
                bvec = jnp.take_along_axis(
                    val16, jnp.full((LANES,), e_loc, jnp.int32), axis=0)
                e = g * LANES + e_loc
                for j in range(DH // LANES):
                    w = rowsb[e, pl.ds(j * LANES, LANES)]
                    lo = lax.bitcast_convert_type(w << 16, jnp.float32)
                    hi = lax.bitcast_convert_type(w & jnp.int32(HI_MASK), jnp.float32)
                    scl[e, pl.ds(j * LANES, LANES)] = lo * bvec
                    scl[e, pl.ds(DH + j * LANES, LANES)] = hi * bvec
            return 0
        lax.fori_loop(0, CHUNK // LANES, _group, 0)

    _issue_col(0, 0)
    _issue_col(1, 1)
    _wait_col(0, 0)
    _issue(0, 0)

    def _pair(i, _):
        for b in range(2):
            k = i * 2 + b
            _wait(k, b)

            @pl.when(k + 2 < NCH)
            def _prefetch_col():
                _issue_col(k + 2, b)

            @pl.when(k + 1 < NCH)
            def _prefetch():
                _wait_col(k + 1, 1 - b)
                _issue(k + 1, 1 - b)
            _unpack_scale(b)
            pltpu.sync_copy(scl, acc.at[row_b[b]], add=True)
        return 0
    lax.fori_loop(0, NCH // 2, _pair, 0)
    plsc.subcore_barrier()

    pltpu.sync_copy(acc.at[pl.ds(rbase, ROWS_PER_TILE)],
                    out_hbm.at[c, pl.ds(rbase, ROWS_PER_TILE)])
    @pl.when(s == last)
    def _write_rem():
        pltpu.sync_copy(acc.at[pl.ds(body, rem)], out_hbm.at[c, pl.ds(body, rem)])


@functools.partial(
    pl.kernel,
    out_type=jax.ShapeDtypeStruct((NUM_CORES, N_NODES, D_FEAT), jnp.float32),
    mesh=plsc.VectorSubcoreMesh(core_axis_name="c", subcore_axis_name="s"),
    compiler_params=pltpu.CompilerParams(use_tc_tiling_on_sc=False),
    scratch_types=[
        pltpu.VMEM((CHUNK,), jnp.int32),
        pltpu.VMEM((CHUNK,), jnp.int32),
        pltpu.VMEM((CHUNK,), jnp.int32),
        pltpu.VMEM((CHUNK,), jnp.int32),
        pltpu.VMEM((CHUNK,), jnp.float32),
        pltpu.VMEM((CHUNK,), jnp.float32),
        pltpu.VMEM((CHUNK, DH), jnp.int32),
        pltpu.VMEM((CHUNK, DH), jnp.int32),
        pltpu.VMEM((CHUNK, D_FEAT), jnp.float32),
        pltpu.VMEM_SHARED((N_NODES, DH), jnp.int32),
        pltpu.VMEM_SHARED((N_NODES, D_FEAT), jnp.float32),
        pltpu.SemaphoreType.DMA,
        pltpu.SemaphoreType.DMA,
        pltpu.SemaphoreType.DMA,
        pltpu.SemaphoreType.DMA,
        pltpu.SemaphoreType.DMA,
        pltpu.SemaphoreType.DMA,
    ],
)
def _sc_spmm(row_hbm, col_hbm, val_hbm, emb_hbm, out_hbm, *scratch):
    _sc_body(row_hbm, col_hbm, val_hbm, emb_hbm, out_hbm, *scratch)


def _combine_body(p_ref, o_ref):
    x = p_ref[0] + p_ref[1]
    o_ref[...] = jnp.where(x >= 0, x, SLOPE * x)


def _combine(partials):
    blk = 1000
    return pl.pallas_call(
        _combine_body,
        grid=(N_NODES // blk,),
        in_specs=[pl.BlockSpec((NUM_CORES, blk, D_FEAT), lambda i: (0, i, 0))],
        out_specs=pl.BlockSpec((blk, D_FEAT), lambda i: (i, 0)),
        out_shape=jax.ShapeDtypeStruct((N_NODES, D_FEAT), jnp.float32),
    )(partials)


def kernel(adj_indices, adj_values, embeds):
    idx = adj_indices.astype(jnp.int32)
    pad2 = ((0, 0), (0, PAD))
    row1 = jnp.pad(idx[0].reshape(NUM_TILES, EDGES_PER_TILE), pad2).reshape(-1)
    col1 = jnp.pad(idx[1].reshape(NUM_TILES, EDGES_PER_TILE), pad2).reshape(-1)
    val1 = jnp.pad(adj_values.reshape(NUM_TILES, EDGES_PER_TILE), pad2).reshape(-1)
    embp = embeds.reshape(N_NODES, 2, DH).transpose(0, 2, 1).astype(jnp.bfloat16)
    embi = jax.lax.bitcast_convert_type(embp, jnp.int32)
    partials = _sc_spmm(row1, col1, val1, embi)
    return _combine(partials)
